# baseline (speedup 1.0000x reference)
import numpy as np

import jax
import jax.numpy as jnp
from jax.experimental import pallas as pl
from jax.experimental.pallas import tpu as pltpu

C1 = 16
C2 = 32
XLANES = 784
N1 = 1024
K2 = 3 * 16 * C1
N2 = 512
Y1LANES = 16 * 16 * C1
Y2LANES = 7 * 8 * C2
FC_HID = 128
FC_OUT = 10
OUT_PAD = 16

_START = [min(max(0, (56 * i - 28) // 128 * 128), XLANES - 256)
          for i in range(14)]

_PARALLEL = pltpu.CompilerParams(dimension_semantics=("parallel",))


def _sel1():
    m = np.zeros((14, 256, 2, 2, 16, 9), np.float32)
    for i in range(14):
        s = _START[i]
        for hp in range(2):
            for dh in range(3):
                r_raw = 2 * i + hp + dh - 1
                if not 0 <= r_raw <= 27:
                    continue
                for parity in range(2):
                    for jp in range(1, 15):
                        for dw in range(3):
                            w_raw = 2 * (jp - 1) + parity + dw - 1
                            if not 0 <= w_raw <= 27:
                                continue
                            k = 28 * r_raw - s + w_raw
                            m[i, k, hp, parity, jp, dh * 3 + dw] = 1.0
    return m.reshape(14 * 256 * 64, 9)


def _sel2():
    m = np.zeros((3, 16, C1, 2, 8, 9, C1), np.float32)
    for dh in range(3):
        for parity in range(2):
            for jp2 in range(7):
                w_out = 2 * jp2 + parity
                for dw in range(3):
                    for ci in range(C1):
                        m[dh, w_out + dw, ci, parity, jp2,
                          dh * 3 + dw, ci] = 1.0
    return m.reshape(3 * 16 * C1 * 2 * 8, 9 * C1)


_SEL1 = _sel1()
_SEL2 = _sel2()
_B1MASK = np.zeros((16, 1), np.float32)
_B1MASK[1:15] = 1.0
_B2MASK = np.zeros((8, 1), np.float32)
_B2MASK[:7] = 1.0


def _fused_cnn_kernel(x_ref, w1_ref, b1_ref, w2_ref, b2_ref,
                      fc1w_ref, fc1b_ref, fc2w_ref, fc2b_ref,
                      o_ref, y1_scr, y2_scr):
    bb = x_ref.shape[0]
    y1_scr[:, pl.ds(0, 256)] = jnp.zeros((bb, 256), jnp.float32)
    y1_scr[:, pl.ds(15 * 256, 256)] = jnp.zeros((bb, 256), jnp.float32)

    b1 = b1_ref[...]
    for i in range(14):
        y = jnp.dot(x_ref[:, pl.ds(_START[i], 256)], w1_ref[i],
                    preferred_element_type=jnp.float32)
        m = jnp.maximum(
            jnp.maximum(y[:, 0:256], y[:, 256:512]),
            jnp.maximum(y[:, 512:768], y[:, 768:1024]))
        y1_scr[:, pl.ds((i + 1) * 256, 256)] = jnp.maximum(m + b1, 0.0)

    w2 = w2_ref[...]
    b2 = b2_ref[...]
    for i in range(7):
        z0 = jnp.dot(y1_scr[:, pl.ds((2 * i) * 256, K2)], w2,
                     preferred_element_type=jnp.float32)
        z1 = jnp.dot(y1_scr[:, pl.ds((2 * i + 1) * 256, K2)], w2,
                     preferred_element_type=jnp.float32)
        m = jnp.maximum(
            jnp.maximum(z0[:, 0:256], z0[:, 256:512]),
            jnp.maximum(z1[:, 0:256], z1[:, 256:512]))
        y2_scr[:, pl.ds(i * 256, 256)] = jnp.maximum(m + b2, 0.0)

    h = jnp.dot(y2_scr[...], fc1w_ref[...],
                preferred_element_type=jnp.float32)
    h = jnp.maximum(h + fc1b_ref[...], 0.0)
    o_ref[...] = (jnp.dot(h, fc2w_ref[...],
                          preferred_element_type=jnp.float32)
                  + fc2b_ref[...])


def _pick_block(batch):
    for cand in (512, 256, 128, 64, 32, 16, 8, 4, 2):
        if batch % cand == 0 and batch // cand >= 2:
            return cand
    return batch


def kernel(x_nchw, conv1_w, conv1_b, conv2_w, conv2_b,
           fc1_w, fc1_b, fc2_w, fc2_b):
    batch = x_nchw.shape[0]
    bb = _pick_block(batch)

    xf = x_nchw.reshape(batch, XLANES)

    w1all = (jnp.asarray(_SEL1) @ conv1_w).reshape(14, 256, N1)
    w2b = (jnp.asarray(_SEL2) @ conv2_w).reshape(K2, N2)
    b1t = (jnp.asarray(_B1MASK) * conv1_b.reshape(1, C1)).reshape(1, 256)
    b2t = (jnp.asarray(_B2MASK) * conv2_b.reshape(1, C2)).reshape(1, 256)
    fc1p = jnp.pad(fc1_w.reshape(7, 7, C2, FC_HID),
                   ((0, 0), (0, 1), (0, 0), (0, 0))).reshape(Y2LANES, FC_HID)
    fc2p = fc2_w[:, :OUT_PAD]
    fc2bp = fc2_b[:, :OUT_PAD]

    out = pl.pallas_call(
        _fused_cnn_kernel,
        out_shape=jax.ShapeDtypeStruct((batch, OUT_PAD), jnp.float32),
        grid=(batch // bb,),
        in_specs=[
            pl.BlockSpec((bb, XLANES), lambda i: (i, 0)),
            pl.BlockSpec((14, 256, N1), lambda i: (0, 0, 0)),
            pl.BlockSpec((1, 256), lambda i: (0, 0)),
            pl.BlockSpec((K2, N2), lambda i: (0, 0)),
            pl.BlockSpec((1, 256), lambda i: (0, 0)),
            pl.BlockSpec((Y2LANES, FC_HID), lambda i: (0, 0)),
            pl.BlockSpec((1, FC_HID), lambda i: (0, 0)),
            pl.BlockSpec((FC_HID, OUT_PAD), lambda i: (0, 0)),
            pl.BlockSpec((1, OUT_PAD), lambda i: (0, 0)),
        ],
        out_specs=pl.BlockSpec((bb, OUT_PAD), lambda i: (i, 0)),
        scratch_shapes=[
            pltpu.VMEM((bb, Y1LANES), jnp.float32),
            pltpu.VMEM((bb, Y2LANES), jnp.float32),
        ],
        compiler_params=_PARALLEL,
    )(xf, w1all, b1t, w2b, b2t, fc1p, fc1_b, fc2p, fc2bp)
    return out[:, :FC_OUT]


# speedup vs baseline: 1.3811x; 1.3811x over previous
import numpy as np

import jax
import jax.numpy as jnp
from jax.experimental import pallas as pl
from jax.experimental.pallas import tpu as pltpu

C1 = 16
C2 = 32
XLANES = 784
N1 = 1024
K2 = 3 * 16 * C1
N2 = 512
Y1LANES = 16 * 16 * C1
Y2LANES = 7 * 8 * C2
FC_HID = 128
FC_OUT = 10
OUT_PAD = 16

_START = [min(max(0, (56 * i - 28) // 128 * 128), XLANES - 256)
          for i in range(14)]

_PARALLEL = pltpu.CompilerParams(dimension_semantics=("parallel",))


def _sel1():
    m = np.zeros((4, 28, 2, 2, 16, 9), np.float32)
    for hp in range(2):
        for dh in range(3):
            dr = hp + dh
            for parity in range(2):
                for jp in range(1, 15):
                    for dw in range(3):
                        w_raw = 2 * (jp - 1) + parity + dw - 1
                        if not 0 <= w_raw <= 27:
                            continue
                        m[dr, w_raw, hp, parity, jp, dh * 3 + dw] = 1.0
    return m.reshape(4 * 28 * 2 * 2 * 16, 9)


def _place1():
    p = np.zeros((14, 256, 4, 28), np.float32)
    for i in range(14):
        for dr in range(4):
            r_raw = 2 * i - 1 + dr
            if not 0 <= r_raw <= 27:
                continue
            for w_raw in range(28):
                p[i, 28 * r_raw - _START[i] + w_raw, dr, w_raw] = 1.0
    return p.reshape(14, 256, 112)


def _sel2():
    m = np.zeros((3, 16, C1, 2, 8, 9, C1), np.float32)
    for dh in range(3):
        for parity in range(2):
            for jp2 in range(7):
                w_out = 2 * jp2 + parity
                for dw in range(3):
                    for ci in range(C1):
                        m[dh, w_out + dw, ci, parity, jp2,
                          dh * 3 + dw, ci] = 1.0
    return m.reshape(3 * 16 * C1 * 2 * 8, 9 * C1)


_SEL1 = _sel1()
_PLACE1 = _place1().reshape(14 * 256, 112)
_SEL2 = _sel2()
_B1MASK = np.zeros((16, 1), np.float32)
_B1MASK[1:15] = 1.0
_B2MASK = np.zeros((8, 1), np.float32)
_B2MASK[:7] = 1.0


def _fused_cnn_kernel(x_ref, w1_ref, b1_ref, w2_ref, b2_ref,
                      fc1w_ref, fc1b_ref, fc2w_ref, fc2b_ref,
                      o_ref, y1_scr, y2_scr):
    bb = x_ref.shape[0]
    y1_scr[:, pl.ds(0, 256)] = jnp.zeros((bb, 256), jnp.float32)
    y1_scr[:, pl.ds(15 * 256, 256)] = jnp.zeros((bb, 256), jnp.float32)

    b1 = b1_ref[...]
    for i in range(14):
        y = jnp.dot(x_ref[:, pl.ds(_START[i], 256)], w1_ref[i],
                    preferred_element_type=jnp.float32)
        m = jnp.maximum(
            jnp.maximum(y[:, 0:256], y[:, 256:512]),
            jnp.maximum(y[:, 512:768], y[:, 768:1024]))
        y1_scr[:, pl.ds((i + 1) * 256, 256)] = jnp.maximum(m + b1, 0.0)

    w2 = w2_ref[...]
    b2 = b2_ref[...]
    for i in range(7):
        z0 = jnp.dot(y1_scr[:, pl.ds((2 * i) * 256, K2)], w2,
                     preferred_element_type=jnp.float32)
        z1 = jnp.dot(y1_scr[:, pl.ds((2 * i + 1) * 256, K2)], w2,
                     preferred_element_type=jnp.float32)
        m = jnp.maximum(
            jnp.maximum(z0[:, 0:256], z0[:, 256:512]),
            jnp.maximum(z1[:, 0:256], z1[:, 256:512]))
        y2_scr[:, pl.ds(i * 256, 256)] = jnp.maximum(m + b2, 0.0)

    h = jnp.dot(y2_scr[...], fc1w_ref[...],
                preferred_element_type=jnp.float32)
    h = jnp.maximum(h + fc1b_ref[...], 0.0)
    o_ref[...] = (jnp.dot(h, fc2w_ref[...],
                          preferred_element_type=jnp.float32)
                  + fc2b_ref[...])


def _pick_block(batch):
    for cand in (512, 256, 128, 64, 32, 16, 8, 4, 2):
        if batch % cand == 0 and batch // cand >= 2:
            return cand
    return batch


def kernel(x_nchw, conv1_w, conv1_b, conv2_w, conv2_b,
           fc1_w, fc1_b, fc2_w, fc2_b):
    batch = x_nchw.shape[0]
    bb = _pick_block(batch)

    xf = x_nchw.reshape(batch, XLANES)

    w1c = (jnp.asarray(_SEL1) @ conv1_w).reshape(112, 64 * C1)
    w1all = (jnp.asarray(_PLACE1) @ w1c).reshape(14, 256, N1)
    w2b = (jnp.asarray(_SEL2) @ conv2_w).reshape(K2, N2)
    b1t = (jnp.asarray(_B1MASK) * conv1_b.reshape(1, C1)).reshape(1, 256)
    b2t = (jnp.asarray(_B2MASK) * conv2_b.reshape(1, C2)).reshape(1, 256)
    fc1p = jnp.pad(fc1_w.reshape(7, 7, C2, FC_HID),
                   ((0, 0), (0, 1), (0, 0), (0, 0))).reshape(Y2LANES, FC_HID)
    fc2p = fc2_w[:, :OUT_PAD]
    fc2bp = fc2_b[:, :OUT_PAD]

    out = pl.pallas_call(
        _fused_cnn_kernel,
        out_shape=jax.ShapeDtypeStruct((batch, OUT_PAD), jnp.float32),
        grid=(batch // bb,),
        in_specs=[
            pl.BlockSpec((bb, XLANES), lambda i: (i, 0)),
            pl.BlockSpec((14, 256, N1), lambda i: (0, 0, 0)),
            pl.BlockSpec((1, 256), lambda i: (0, 0)),
            pl.BlockSpec((K2, N2), lambda i: (0, 0)),
            pl.BlockSpec((1, 256), lambda i: (0, 0)),
            pl.BlockSpec((Y2LANES, FC_HID), lambda i: (0, 0)),
            pl.BlockSpec((1, FC_HID), lambda i: (0, 0)),
            pl.BlockSpec((FC_HID, OUT_PAD), lambda i: (0, 0)),
            pl.BlockSpec((1, OUT_PAD), lambda i: (0, 0)),
        ],
        out_specs=pl.BlockSpec((bb, OUT_PAD), lambda i: (i, 0)),
        scratch_shapes=[
            pltpu.VMEM((bb, Y1LANES), jnp.float32),
            pltpu.VMEM((bb, Y2LANES), jnp.float32),
        ],
        compiler_params=_PARALLEL,
    )(xf, w1all, b1t, w2b, b2t, fc1p, fc1_b, fc2p, fc2bp)
    return out[:, :FC_OUT]


# speedup vs baseline: 1.3960x; 1.0108x over previous
import numpy as np

import jax
import jax.numpy as jnp
from jax.experimental import pallas as pl
from jax.experimental.pallas import tpu as pltpu

C1 = 16
C2 = 32
XLANES = 784
N1 = 1024
K2 = 3 * 16 * C1
N2 = 512
Y1LANES = 16 * 16 * C1
Y2LANES = 7 * 8 * C2
FC_HID = 128
FC_OUT = 10
OUT_PAD = 16

_START = [min(max(0, (56 * i - 28) // 128 * 128), XLANES - 256)
          for i in range(14)]

_PARALLEL = pltpu.CompilerParams(dimension_semantics=("parallel",))


def _sel1():
    m = np.zeros((4, 28, 2, 2, 16, 9), np.float32)
    for hp in range(2):
        for dh in range(3):
            dr = hp + dh
            for parity in range(2):
                for jp in range(1, 15):
                    for dw in range(3):
                        w_raw = 2 * (jp - 1) + parity + dw - 1
                        if not 0 <= w_raw <= 27:
                            continue
                        m[dr, w_raw, hp, parity, jp, dh * 3 + dw] = 1.0
    return m.reshape(4 * 28 * 2 * 2 * 16, 9)


def _place1():
    p = np.zeros((14, 256, 4, 28), np.float32)
    for i in range(14):
        for dr in range(4):
            r_raw = 2 * i - 1 + dr
            if not 0 <= r_raw <= 27:
                continue
            for w_raw in range(28):
                p[i, 28 * r_raw - _START[i] + w_raw, dr, w_raw] = 1.0
    return p.reshape(14, 256, 112)


def _sel2():
    m = np.zeros((3, 16, C1, 2, 8, 9, C1), np.float32)
    for dh in range(3):
        for parity in range(2):
            for jp2 in range(7):
                w_out = 2 * jp2 + parity
                for dw in range(3):
                    for ci in range(C1):
                        m[dh, w_out + dw, ci, parity, jp2,
                          dh * 3 + dw, ci] = 1.0
    return m.reshape(3 * 16 * C1 * 2 * 8, 9 * C1)


_SEL1 = _sel1()
_PLACE1 = _place1().reshape(14 * 256, 112)
_SEL2 = _sel2()
_B1MASK = np.zeros((16, 1), np.float32)
_B1MASK[1:15] = 1.0
_B2MASK = np.zeros((8, 1), np.float32)
_B2MASK[:7] = 1.0


def _fused_cnn_kernel(x_ref, w1_ref, b1_ref, w2_ref, b2_ref,
                      fc1w_ref, fc1b_ref, fc2w_ref, fc2b_ref,
                      o_ref, y1_scr, y2_scr):
    bb = x_ref.shape[0]
    y1_scr[:, pl.ds(0, 256)] = jnp.zeros((bb, 256), jnp.float32)
    y1_scr[:, pl.ds(15 * 256, 256)] = jnp.zeros((bb, 256), jnp.float32)

    b1 = b1_ref[...]
    for i in range(14):
        y = jnp.dot(x_ref[:, pl.ds(_START[i], 256)], w1_ref[i],
                    preferred_element_type=jnp.float32)
        m = jnp.maximum(
            jnp.maximum(y[:, 0:256], y[:, 256:512]),
            jnp.maximum(y[:, 512:768], y[:, 768:1024]))
        y1_scr[:, pl.ds((i + 1) * 256, 256)] = jnp.maximum(m + b1, 0.0)

    w2 = w2_ref[...]
    b2 = b2_ref[...]
    for i in range(7):
        z0 = jnp.dot(y1_scr[:, pl.ds((2 * i) * 256, K2)], w2,
                     preferred_element_type=jnp.float32)
        z1 = jnp.dot(y1_scr[:, pl.ds((2 * i + 1) * 256, K2)], w2,
                     preferred_element_type=jnp.float32)
        m = jnp.maximum(
            jnp.maximum(z0[:, 0:256], z0[:, 256:512]),
            jnp.maximum(z1[:, 0:256], z1[:, 256:512]))
        y2_scr[:, pl.ds(i * 256, 256)] = jnp.maximum(m + b2, 0.0)

    h = jnp.dot(y2_scr[...], fc1w_ref[...],
                preferred_element_type=jnp.float32)
    h = jnp.maximum(h + fc1b_ref[...], 0.0)
    o_ref[...] = (jnp.dot(h, fc2w_ref[...],
                          preferred_element_type=jnp.float32)
                  + fc2b_ref[...])


def _pick_block(batch):
    for cand in (1024, 512, 256, 128, 64, 32, 16, 8, 4, 2):
        if batch % cand == 0 and batch // cand >= 2:
            return cand
    return batch


def kernel(x_nchw, conv1_w, conv1_b, conv2_w, conv2_b,
           fc1_w, fc1_b, fc2_w, fc2_b):
    batch = x_nchw.shape[0]
    bb = _pick_block(batch)

    xf = x_nchw.reshape(batch, XLANES)

    w1c = (jnp.asarray(_SEL1) @ conv1_w).reshape(112, 64 * C1)
    w1all = (jnp.asarray(_PLACE1) @ w1c).reshape(14, 256, N1)
    w2b = (jnp.asarray(_SEL2) @ conv2_w).reshape(K2, N2)
    b1t = (jnp.asarray(_B1MASK) * conv1_b.reshape(1, C1)).reshape(1, 256)
    b2t = (jnp.asarray(_B2MASK) * conv2_b.reshape(1, C2)).reshape(1, 256)
    fc1p = jnp.pad(fc1_w.reshape(7, 7, C2, FC_HID),
                   ((0, 0), (0, 1), (0, 0), (0, 0))).reshape(Y2LANES, FC_HID)
    fc2p = fc2_w[:, :OUT_PAD]
    fc2bp = fc2_b[:, :OUT_PAD]

    out = pl.pallas_call(
        _fused_cnn_kernel,
        out_shape=jax.ShapeDtypeStruct((batch, OUT_PAD), jnp.float32),
        grid=(batch // bb,),
        in_specs=[
            pl.BlockSpec((bb, XLANES), lambda i: (i, 0)),
            pl.BlockSpec((14, 256, N1), lambda i: (0, 0, 0)),
            pl.BlockSpec((1, 256), lambda i: (0, 0)),
            pl.BlockSpec((K2, N2), lambda i: (0, 0)),
            pl.BlockSpec((1, 256), lambda i: (0, 0)),
            pl.BlockSpec((Y2LANES, FC_HID), lambda i: (0, 0)),
            pl.BlockSpec((1, FC_HID), lambda i: (0, 0)),
            pl.BlockSpec((FC_HID, OUT_PAD), lambda i: (0, 0)),
            pl.BlockSpec((1, OUT_PAD), lambda i: (0, 0)),
        ],
        out_specs=pl.BlockSpec((bb, OUT_PAD), lambda i: (i, 0)),
        scratch_shapes=[
            pltpu.VMEM((bb, Y1LANES), jnp.float32),
            pltpu.VMEM((bb, Y2LANES), jnp.float32),
        ],
        compiler_params=_PARALLEL,
    )(xf, w1all, b1t, w2b, b2t, fc1p, fc1_b, fc2p, fc2bp)
    return out[:, :FC_OUT]


# speedup vs baseline: 1.4150x; 1.0136x over previous
import numpy as np

import jax
import jax.numpy as jnp
from jax.experimental import pallas as pl
from jax.experimental.pallas import tpu as pltpu

C1 = 16
C2 = 32
XLANES = 784
N1 = 1024
K2 = 3 * 16 * C1
N2 = 512
Y1LANES = 16 * 16 * C1
Y2LANES = 7 * 8 * C2
FC_HID = 128
FC_OUT = 10
OUT_PAD = 16

_START = [min(max(0, (56 * i - 28) // 128 * 128), XLANES - 256)
          for i in range(14)]

_PARALLEL = pltpu.CompilerParams(dimension_semantics=("parallel",))


def _sel1():
    m = np.zeros((4, 28, 2, 2, 16, 9), np.float32)
    for hp in range(2):
        for dh in range(3):
            dr = hp + dh
            for parity in range(2):
                for jp in range(1, 15):
                    for dw in range(3):
                        w_raw = 2 * (jp - 1) + parity + dw - 1
                        if not 0 <= w_raw <= 27:
                            continue
                        m[dr, w_raw, hp, parity, jp, dh * 3 + dw] = 1.0
    return m.reshape(4 * 28 * 2 * 2 * 16, 9)


def _place1():
    p = np.zeros((14, 256, 4, 28), np.float32)
    for i in range(14):
        for dr in range(4):
            r_raw = 2 * i - 1 + dr
            if not 0 <= r_raw <= 27:
                continue
            for w_raw in range(28):
                p[i, 28 * r_raw - _START[i] + w_raw, dr, w_raw] = 1.0
    return p.reshape(14, 256, 112)


def _sel2():
    m = np.zeros((3, 16, C1, 2, 8, 9, C1), np.float32)
    for dh in range(3):
        for parity in range(2):
            for jp2 in range(7):
                w_out = 2 * jp2 + parity
                for dw in range(3):
                    for ci in range(C1):
                        m[dh, w_out + dw, ci, parity, jp2,
                          dh * 3 + dw, ci] = 1.0
    return m.reshape(3 * 16 * C1 * 2 * 8, 9 * C1)


_SEL1 = _sel1()
_PLACE1 = _place1().reshape(14 * 256, 112)
_SEL2 = _sel2()
_B1MASK = np.zeros((16, 1), np.float32)
_B1MASK[1:15] = 1.0
_B2MASK = np.zeros((8, 1), np.float32)
_B2MASK[:7] = 1.0


def _fused_cnn_kernel(x_ref, w1_ref, b1_ref, w2_ref, b2_ref,
                      fc1w_ref, fc1b_ref, fc2w_ref, fc2b_ref,
                      o_ref, x_scr, y1_scr, y2_scr):
    bb = x_ref.shape[0]
    x_scr[...] = x_ref[...].astype(jnp.bfloat16)
    y1_scr[:, pl.ds(0, 256)] = jnp.zeros((bb, 256), jnp.bfloat16)
    y1_scr[:, pl.ds(15 * 256, 256)] = jnp.zeros((bb, 256), jnp.bfloat16)

    b1 = b1_ref[...]
    for i in range(14):
        y = jnp.dot(x_scr[:, pl.ds(_START[i], 256)], w1_ref[i],
                    preferred_element_type=jnp.float32)
        m = jnp.maximum(
            jnp.maximum(y[:, 0:256], y[:, 256:512]),
            jnp.maximum(y[:, 512:768], y[:, 768:1024]))
        y1_scr[:, pl.ds((i + 1) * 256, 256)] = (
            jnp.maximum(m + b1, 0.0).astype(jnp.bfloat16))

    w2 = w2_ref[...]
    b2 = b2_ref[...]
    for i in range(7):
        z0 = jnp.dot(y1_scr[:, pl.ds((2 * i) * 256, K2)], w2,
                     preferred_element_type=jnp.float32)
        z1 = jnp.dot(y1_scr[:, pl.ds((2 * i + 1) * 256, K2)], w2,
                     preferred_element_type=jnp.float32)
        m = jnp.maximum(
            jnp.maximum(z0[:, 0:256], z0[:, 256:512]),
            jnp.maximum(z1[:, 0:256], z1[:, 256:512]))
        y2_scr[:, pl.ds(i * 256, 256)] = (
            jnp.maximum(m + b2, 0.0).astype(jnp.bfloat16))

    h = jnp.dot(y2_scr[...], fc1w_ref[...],
                preferred_element_type=jnp.float32)
    h = jnp.maximum(h + fc1b_ref[...], 0.0).astype(jnp.bfloat16)
    o_ref[...] = (jnp.dot(h, fc2w_ref[...],
                          preferred_element_type=jnp.float32)
                  + fc2b_ref[...])


def _pick_block(batch):
    for cand in (1024, 512, 256, 128, 64, 32, 16, 8, 4, 2):
        if batch % cand == 0 and batch // cand >= 2:
            return cand
    return batch


def kernel(x_nchw, conv1_w, conv1_b, conv2_w, conv2_b,
           fc1_w, fc1_b, fc2_w, fc2_b):
    batch = x_nchw.shape[0]
    bb = _pick_block(batch)

    xf = x_nchw.reshape(batch, XLANES)

    w1c = (jnp.asarray(_SEL1) @ conv1_w).reshape(112, 64 * C1)
    w1all = (jnp.asarray(_PLACE1) @ w1c).reshape(14, 256, N1)
    w1all = w1all.astype(jnp.bfloat16)
    w2b = (jnp.asarray(_SEL2) @ conv2_w).astype(jnp.bfloat16).reshape(K2, N2)
    b1t = (jnp.asarray(_B1MASK) * conv1_b.reshape(1, C1)).reshape(1, 256)
    b2t = (jnp.asarray(_B2MASK) * conv2_b.reshape(1, C2)).reshape(1, 256)
    fc1p = jnp.pad(fc1_w.reshape(7, 7, C2, FC_HID),
                   ((0, 0), (0, 1), (0, 0), (0, 0))).reshape(
                       Y2LANES, FC_HID).astype(jnp.bfloat16)
    fc2p = fc2_w[:, :OUT_PAD].astype(jnp.bfloat16)
    fc2bp = fc2_b[:, :OUT_PAD]

    out = pl.pallas_call(
        _fused_cnn_kernel,
        out_shape=jax.ShapeDtypeStruct((batch, OUT_PAD), jnp.float32),
        grid=(batch // bb,),
        in_specs=[
            pl.BlockSpec((bb, XLANES), lambda i: (i, 0)),
            pl.BlockSpec((14, 256, N1), lambda i: (0, 0, 0)),
            pl.BlockSpec((1, 256), lambda i: (0, 0)),
            pl.BlockSpec((K2, N2), lambda i: (0, 0)),
            pl.BlockSpec((1, 256), lambda i: (0, 0)),
            pl.BlockSpec((Y2LANES, FC_HID), lambda i: (0, 0)),
            pl.BlockSpec((1, FC_HID), lambda i: (0, 0)),
            pl.BlockSpec((FC_HID, OUT_PAD), lambda i: (0, 0)),
            pl.BlockSpec((1, OUT_PAD), lambda i: (0, 0)),
        ],
        out_specs=pl.BlockSpec((bb, OUT_PAD), lambda i: (i, 0)),
        scratch_shapes=[
            pltpu.VMEM((bb, XLANES), jnp.bfloat16),
            pltpu.VMEM((bb, Y1LANES), jnp.bfloat16),
            pltpu.VMEM((bb, Y2LANES), jnp.bfloat16),
        ],
        compiler_params=_PARALLEL,
    )(xf, w1all, b1t, w2b, b2t, fc1p, fc1_b, fc2p, fc2bp)
    return out[:, :FC_OUT]


# speedup vs baseline: 1.4423x; 1.0193x over previous
import numpy as np

import jax
import jax.numpy as jnp
from jax.experimental import pallas as pl
from jax.experimental.pallas import tpu as pltpu

C1 = 16
C2 = 32
XLANES = 784
N1 = 1024
K2 = 3 * 16 * C1
N2 = 512
Y1LANES = 16 * 16 * C1
Y2LANES = 7 * 8 * C2
FC_HID = 128
FC_OUT = 10
OUT_PAD = 16

_START = [min(max(0, (56 * i - 28) // 128 * 128), XLANES - 256)
          for i in range(14)]

_PARALLEL = pltpu.CompilerParams(dimension_semantics=("parallel",))


def _s2():
    m = np.zeros((4, 28, 64, 9), np.float32)
    for hp in range(2):
        for dh in range(3):
            dr = hp + dh
            for parity in range(2):
                for jp in range(1, 15):
                    g = (hp * 2 + parity) * 16 + jp
                    for dw in range(3):
                        w_raw = 2 * (jp - 1) + parity + dw - 1
                        if 0 <= w_raw <= 27:
                            m[dr, w_raw, g, dh * 3 + dw] = 1.0
    return m.reshape(112, 576)


def _place1():
    p = np.zeros((14, 256, 4, 28), np.float32)
    for i in range(14):
        for dr in range(4):
            r_raw = 2 * i - 1 + dr
            if not 0 <= r_raw <= 27:
                continue
            for w_raw in range(28):
                p[i, 28 * r_raw - _START[i] + w_raw, dr, w_raw] = 1.0
    return p.reshape(14 * 256, 112)


def _a2():
    m = np.zeros((3, 3, 16, C1, 9, C1), np.float32)
    for dw in range(3):
        for dh in range(3):
            for w_in in range(16):
                for ci in range(C1):
                    m[dw, dh, w_in, ci, dh * 3 + dw, ci] = 1.0
    return m.reshape(3, K2, 9 * C1)


def _m2():
    m = np.zeros((3, 3, 16, C1, 2, 8, C2), np.float32)
    for dw in range(3):
        for parity in range(2):
            for jp2 in range(7):
                w_in = 2 * jp2 + parity + dw
                m[dw, :, w_in, :, parity, jp2, :] = 1.0
    return m.reshape(3, K2, N2)


_TILE1 = np.tile(np.eye(16, dtype=np.float32), (1, 64))
_A1G = np.tile(np.eye(9, dtype=np.float32), (64, 1))
_MASKG = np.kron(np.eye(64, dtype=np.float32), np.ones((9, 16), np.float32))
_S2 = _s2()
_PLACE1 = _place1()
_W2TILE = np.tile(np.eye(C2, dtype=np.float32), (1, 16))
_A2 = _a2()
_M2 = _m2()
_T1B = np.zeros((16, 256), np.float32)
for _jp in range(1, 15):
    _T1B[:, _jp * 16:(_jp + 1) * 16] = np.eye(16, dtype=np.float32)
_T2B = np.zeros((32, 256), np.float32)
for _jp2 in range(7):
    _T2B[:, _jp2 * 32:(_jp2 + 1) * 32] = np.eye(32, dtype=np.float32)


def _pack_kernel(w1_ref, w2_ref, b1_ref, b2_ref, fc1w_ref, fc2w_ref,
                 fc2b_ref, tile1_ref, a1g_ref, maskg_ref, s2_ref, place_ref,
                 w2tile_ref, a2_ref, m2_ref, t1b_ref, t2b_ref,
                 w1all_ref, w2b_ref, fc1p_ref, fc2p_ref, b1t_ref, b2t_ref,
                 fc2bt_ref, wg_scr, w1c_scr, w2t_scr):
    f32 = jnp.float32
    w1t = jnp.dot(w1_ref[...], tile1_ref[...], preferred_element_type=f32)
    wg_scr[...] = maskg_ref[...] * jnp.dot(
        a1g_ref[...], w1t, preferred_element_type=f32)
    w1c_scr[...] = jnp.dot(s2_ref[...], wg_scr[...],
                           preferred_element_type=f32)
    for i in range(14):
        w1all_ref[pl.ds(i * 256, 256), :] = jnp.dot(
            place_ref[pl.ds(i * 256, 256), :], w1c_scr[...],
            preferred_element_type=f32).astype(jnp.bfloat16)

    w2t_scr[...] = jnp.dot(w2_ref[...], w2tile_ref[...],
                           preferred_element_type=f32)
    acc = m2_ref[0] * jnp.dot(a2_ref[0], w2t_scr[...],
                              preferred_element_type=f32)
    acc += m2_ref[1] * jnp.dot(a2_ref[1], w2t_scr[...],
                               preferred_element_type=f32)
    acc += m2_ref[2] * jnp.dot(a2_ref[2], w2t_scr[...],
                               preferred_element_type=f32)
    w2b_ref[...] = acc.astype(jnp.bfloat16)

    for i2 in range(7):
        fc1p_ref[pl.ds(i2 * 256, 224), :] = (
            fc1w_ref[pl.ds(i2 * 224, 224), :].astype(jnp.bfloat16))
        fc1p_ref[pl.ds(i2 * 256 + 224, 32), :] = jnp.zeros(
            (32, FC_HID), jnp.bfloat16)

    fc2p_ref[...] = fc2w_ref[:, :OUT_PAD].astype(jnp.bfloat16)
    b1t_ref[...] = jnp.dot(b1_ref[...], t1b_ref[...],
                           preferred_element_type=f32)
    b2t_ref[...] = jnp.dot(b2_ref[...], t2b_ref[...],
                           preferred_element_type=f32)
    fc2bt_ref[...] = fc2b_ref[:, :OUT_PAD]


def _pack_weights(conv1_w, conv2_w, conv1_b, conv2_b, fc1_w, fc2_w, fc2_b):
    bf16 = jnp.bfloat16
    f32 = jnp.float32
    whole = lambda *shape: pl.BlockSpec(shape, lambda: tuple(0 for _ in shape))
    return pl.pallas_call(
        _pack_kernel,
        out_shape=(
            jax.ShapeDtypeStruct((14 * 256, N1), bf16),
            jax.ShapeDtypeStruct((K2, N2), bf16),
            jax.ShapeDtypeStruct((Y2LANES, FC_HID), bf16),
            jax.ShapeDtypeStruct((FC_HID, OUT_PAD), bf16),
            jax.ShapeDtypeStruct((1, 256), f32),
            jax.ShapeDtypeStruct((1, 256), f32),
            jax.ShapeDtypeStruct((1, OUT_PAD), f32),
        ),
        in_specs=[
            whole(9, C1), whole(9 * C1, C2), whole(1, C1), whole(1, C2),
            whole(7 * 7 * C2, FC_HID), whole(FC_HID, FC_HID),
            whole(1, FC_HID),
            whole(16, N1), whole(576, 9), whole(576, N1), whole(112, 576),
            whole(14 * 256, 112), whole(C2, N2), whole(3, K2, 9 * C1),
            whole(3, K2, N2), whole(16, 256), whole(C2, 256),
        ],
        out_specs=(
            whole(14 * 256, N1), whole(K2, N2), whole(Y2LANES, FC_HID),
            whole(FC_HID, OUT_PAD), whole(1, 256), whole(1, 256),
            whole(1, OUT_PAD),
        ),
        scratch_shapes=[
            pltpu.VMEM((576, N1), f32),
            pltpu.VMEM((112, N1), f32),
            pltpu.VMEM((9 * C1, N2), f32),
        ],
    )(conv1_w, conv2_w, conv1_b, conv2_b, fc1_w, fc2_w, fc2_b,
      jnp.asarray(_TILE1), jnp.asarray(_A1G), jnp.asarray(_MASKG),
      jnp.asarray(_S2), jnp.asarray(_PLACE1), jnp.asarray(_W2TILE),
      jnp.asarray(_A2), jnp.asarray(_M2), jnp.asarray(_T1B),
      jnp.asarray(_T2B))


def _fused_cnn_kernel(x_ref, w1_ref, b1_ref, w2_ref, b2_ref,
                      fc1w_ref, fc1b_ref, fc2w_ref, fc2b_ref,
                      o_ref, x_scr, y1_scr, y2_scr):
    bb = x_ref.shape[0]
    x_scr[...] = x_ref[...].astype(jnp.bfloat16)
    y1_scr[:, pl.ds(0, 256)] = jnp.zeros((bb, 256), jnp.bfloat16)
    y1_scr[:, pl.ds(15 * 256, 256)] = jnp.zeros((bb, 256), jnp.bfloat16)

    b1 = b1_ref[...]
    for i in range(14):
        y = jnp.dot(x_scr[:, pl.ds(_START[i], 256)], w1_ref[i],
                    preferred_element_type=jnp.float32)
        m = jnp.maximum(
            jnp.maximum(y[:, 0:256], y[:, 256:512]),
            jnp.maximum(y[:, 512:768], y[:, 768:1024]))
        y1_scr[:, pl.ds((i + 1) * 256, 256)] = (
            jnp.maximum(m + b1, 0.0).astype(jnp.bfloat16))

    w2 = w2_ref[...]
    b2 = b2_ref[...]
    for i in range(7):
        z0 = jnp.dot(y1_scr[:, pl.ds((2 * i) * 256, K2)], w2,
                     preferred_element_type=jnp.float32)
        z1 = jnp.dot(y1_scr[:, pl.ds((2 * i + 1) * 256, K2)], w2,
                     preferred_element_type=jnp.float32)
        m = jnp.maximum(
            jnp.maximum(z0[:, 0:256], z0[:, 256:512]),
            jnp.maximum(z1[:, 0:256], z1[:, 256:512]))
        y2_scr[:, pl.ds(i * 256, 256)] = (
            jnp.maximum(m + b2, 0.0).astype(jnp.bfloat16))

    h = jnp.dot(y2_scr[...], fc1w_ref[...],
                preferred_element_type=jnp.float32)
    h = jnp.maximum(h + fc1b_ref[...], 0.0).astype(jnp.bfloat16)
    o_ref[...] = (jnp.dot(h, fc2w_ref[...],
                          preferred_element_type=jnp.float32)
                  + fc2b_ref[...])


def _pick_block(batch):
    for cand in (1024, 512, 256, 128, 64, 32, 16, 8, 4, 2):
        if batch % cand == 0 and batch // cand >= 2:
            return cand
    return batch


def kernel(x_nchw, conv1_w, conv1_b, conv2_w, conv2_b,
           fc1_w, fc1_b, fc2_w, fc2_b):
    batch = x_nchw.shape[0]
    bb = _pick_block(batch)

    xf = x_nchw.reshape(batch, XLANES)
    (w1all, w2b, fc1p, fc2p, b1t, b2t, fc2bt) = _pack_weights(
        conv1_w, conv2_w, conv1_b, conv2_b, fc1_w, fc2_w, fc2_b)
    w1all = w1all.reshape(14, 256, N1)

    out = pl.pallas_call(
        _fused_cnn_kernel,
        out_shape=jax.ShapeDtypeStruct((batch, OUT_PAD), jnp.float32),
        grid=(batch // bb,),
        in_specs=[
            pl.BlockSpec((bb, XLANES), lambda i: (i, 0)),
            pl.BlockSpec((14, 256, N1), lambda i: (0, 0, 0)),
            pl.BlockSpec((1, 256), lambda i: (0, 0)),
            pl.BlockSpec((K2, N2), lambda i: (0, 0)),
            pl.BlockSpec((1, 256), lambda i: (0, 0)),
            pl.BlockSpec((Y2LANES, FC_HID), lambda i: (0, 0)),
            pl.BlockSpec((1, FC_HID), lambda i: (0, 0)),
            pl.BlockSpec((FC_HID, OUT_PAD), lambda i: (0, 0)),
            pl.BlockSpec((1, OUT_PAD), lambda i: (0, 0)),
        ],
        out_specs=pl.BlockSpec((bb, OUT_PAD), lambda i: (i, 0)),
        scratch_shapes=[
            pltpu.VMEM((bb, XLANES), jnp.bfloat16),
            pltpu.VMEM((bb, Y1LANES), jnp.bfloat16),
            pltpu.VMEM((bb, Y2LANES), jnp.bfloat16),
        ],
        compiler_params=_PARALLEL,
    )(xf, w1all, b1t, w2b, b2t, fc1p, fc1_b, fc2p, fc2bt)
    return out[:, :FC_OUT]
